# baseline (speedup 1.0000x reference)
import jax
import jax.numpy as jnp
from jax import lax
from jax.experimental import pallas as pl
from jax.experimental.pallas import tpu as pltpu
from jax.experimental.pallas import tpu_sc as plsc

N = 50000
E = 800000
H = 128
NB = 4
QN = N // NB
QROWS = 12544
NP = NB * QROWS
NBLK = QROWS // 128
CAPQ = 262144
CAPH = CAPQ // 2
EP = NB * CAPQ
EPP = EP + 64
C = 128
CE = 112
AROWS = 12512
ABLK = AROWS // 32
EHALF = E // 2
ETILE = E // 32
NBATCH = (ETILE + C - 1) // C


def _iota16():
    return lax.broadcasted_iota(jnp.int32, (16,), 0)


def _prefix16(m):
    iota = _iota16()
    v = jnp.where(m, 1, 0)
    for k in (1, 2, 4, 8):
        sh = v.at[jnp.maximum(iota - k, 0)].get(mode="promise_in_bounds")
        v = v + jnp.where(iota >= k, sh, 0)
    return v


def _bucket_body(srcE, dstE, px, py, pz,
                 srcp_o, dstp_o, d2p_o, cnts_o,
                 sv, dv, d2s, pidx, gx, gy, gz, hx, hy, hz, c16, csm, sem):
    c_idx = lax.axis_index("c")
    s_idx = lax.axis_index("s")
    iota = _iota16()

    @pl.when(s_idx == 0)
    def _():
        for b in range(4):
            csm[b] = 0
    plsc.subcore_barrier()

    tilebase = c_idx * EHALF + s_idx * ETILE

    def _batch(i, _):
        base = pl.multiple_of(tilebase + i * C, 8)
        pltpu.sync_copy(srcE.at[pl.ds(base, C)], sv)
        pltpu.sync_copy(dstE.at[pl.ds(base, C)], dv)
        d1 = pltpu.async_copy(px.at[sv], gx, sem)
        d2_ = pltpu.async_copy(py.at[sv], gy, sem)
        d3 = pltpu.async_copy(pz.at[sv], gz, sem)
        d4 = pltpu.async_copy(px.at[dv], hx, sem)
        d5 = pltpu.async_copy(py.at[dv], hy, sem)
        d6 = pltpu.async_copy(pz.at[dv], hz, sem)
        for d in (d1, d2_, d3, d4, d5, d6):
            d.wait()
        bvs = []
        pcs = []
        tl0 = i * C
        for kv in range(8):
            sl = pl.ds(16 * kv, 16)
            ax = gx[sl] - hx[sl]
            ay = gy[sl] - hy[sl]
            az = gz[sl] - hz[sl]
            d2s[sl] = ax * ax + ay * ay + az * az
            dvv = dv[sl]
            bv = (jnp.where(dvv >= QN, 1, 0)
                  + jnp.where(dvv >= 2 * QN, 1, 0)
                  + jnp.where(dvv >= 3 * QN, 1, 0))
            tl = iota + (tl0 + 16 * kv)
            bv = jnp.where(tl < ETILE, bv, 4)
            bvs.append(bv)
            row = []
            for b in range(4):
                row.append(_prefix16(bv == b)[15])
            pcs.append(row)
        gbase = []
        for b in range(4):
            nb_b = pcs[0][b]
            for kv in range(1, 8):
                nb_b = nb_b + pcs[kv][b]
            old = plsc.fetch_and_add(csm.at[b], nb_b, subcore_id=0)
            gbase.append(b * CAPQ + c_idx * CAPH + old)
        pre = [0, 0, 0, 0]
        for kv in range(8):
            sl = pl.ds(16 * kv, 16)
            bv = bvs[kv]
            pos = jnp.full((16,), EP, jnp.int32)
            for b in range(4):
                m = bv == b
                cs = _prefix16(m)
                pos = jnp.where(m, (gbase[b] + pre[b] - 1) + cs, pos)
                pre[b] = pre[b] + pcs[kv][b]
            pidx[sl] = pos
            svv = sv[sl]
            sv[sl] = svv + 44 * (jnp.where(svv >= QN, 1, 0)
                                 + jnp.where(svv >= 2 * QN, 1, 0)
                                 + jnp.where(svv >= 3 * QN, 1, 0))
        e1 = pltpu.async_copy(sv, srcp_o.at[pidx], sem)
        e2 = pltpu.async_copy(dv, dstp_o.at[pidx], sem)
        e3 = pltpu.async_copy(d2s, d2p_o.at[pidx], sem)
        e1.wait()
        e2.wait()
        e3.wait()
        return 0

    lax.fori_loop(0, NBATCH, _batch, 0)
    plsc.subcore_barrier()

    @pl.when(s_idx == 0)
    def _():
        v = jnp.zeros((16,), jnp.int32)
        for b in range(4):
            v = jnp.where(iota == b, csm[b], v)
        c16[...] = v
        pltpu.sync_copy(c16, cnts_o.at[c_idx])


def _bucket_edges(srcE, dstE, px, py, pz):
    mesh = plsc.VectorSubcoreMesh(core_axis_name="c", subcore_axis_name="s")
    f = pl.kernel(
        _bucket_body,
        out_type=(
            jax.ShapeDtypeStruct((EPP,), jnp.int32),
            jax.ShapeDtypeStruct((EPP,), jnp.int32),
            jax.ShapeDtypeStruct((EPP,), jnp.float32),
            jax.ShapeDtypeStruct((2, 16), jnp.int32),
        ),
        mesh=mesh,
        scratch_types=[
            pltpu.VMEM((C,), jnp.int32),
            pltpu.VMEM((C,), jnp.int32),
            pltpu.VMEM((C,), jnp.float32),
            pltpu.VMEM((C,), jnp.int32),
            pltpu.VMEM((C,), jnp.float32),
            pltpu.VMEM((C,), jnp.float32),
            pltpu.VMEM((C,), jnp.float32),
            pltpu.VMEM((C,), jnp.float32),
            pltpu.VMEM((C,), jnp.float32),
            pltpu.VMEM((C,), jnp.float32),
            pltpu.VMEM((16,), jnp.int32),
            pltpu.SMEM((8,), jnp.int32),
            pltpu.SemaphoreType.DMA,
        ],
    )
    return f(srcE, dstE, px, py, pz)


def _edge_layer_body(h_hbm, srcp_hbm, dstp_hbm, d2p_hbm, wr_hbm, br_hbm,
                     cnts_hbm, agg_hbm,
                     c0_v, c1_v, wr_v, br_v, sidx_v, dstl_v, d2_v, rows_v,
                     sidx_w, dstl_w, d2_w, rows_w, acc_sh, sem, semb, semm):
    c_idx = lax.axis_index("c")
    s_idx = lax.axis_index("s")
    pltpu.sync_copy(cnts_hbm.at[0], c0_v)
    pltpu.sync_copy(cnts_hbm.at[1], c1_v)
    pltpu.sync_copy(wr_hbm, wr_v)
    pltpu.sync_copy(br_hbm, br_v)
    c0 = c0_v[...]
    c1 = c1_v[...]
    iota = _iota16()
    jsel = s_idx >> 3
    t8 = s_idx & 7

    def _round(r, _carry):
        q = c_idx + 2 * r
        cnt0 = jnp.where(r == 0,
                         jnp.where(c_idx == 0, c0[0], c0[1]),
                         jnp.where(c_idx == 0, c0[2], c0[3]))
        cnt1 = jnp.where(r == 0,
                         jnp.where(c_idx == 0, c1[0], c1[1]),
                         jnp.where(c_idx == 0, c1[2], c1[3]))
        cnt = jnp.where(jsel == 0, cnt0, cnt1)
        rb = q * CAPQ + jsel * CAPH
        qnode = q * QN
        share = ((cnt + 8 * CE - 1) // (8 * CE)) * CE
        start = rb + t8 * share
        tend = jnp.minimum(start + share, rb + cnt)
        nch = (jnp.maximum(tend - start, 0) + CE - 1) // CE

        plsc.subcore_barrier()

        def _zrow(j, _):
            for k in range(8):
                rows_v[j, pl.ds(16 * k, 16)] = jnp.zeros((16,), jnp.float32)
            return 0
        lax.fori_loop(0, 32, _zrow, 0)
        for k in range(25):
            zb = s_idx + 16 * k
            @pl.when(zb < ABLK)
            def _():
                pltpu.sync_copy(rows_v.at[pl.ds(0, 32)],
                                acc_sh.at[pl.ds(zb * 32, 32)])
        plsc.subcore_barrier()

        def _sanitize(base, si, dl, dd):
            for kv in range(CE // 16):
                sl = pl.ds(16 * kv, 16)
                posv = iota + (base + 16 * kv)
                valid = posv < tend
                si[sl] = jnp.clip(si[sl], 0, NP - 1)
                dv = jnp.clip(dl[sl] - qnode, 0, QN - 1)
                dl[sl] = jnp.where(valid, dv, QN)
                dd[sl] = jnp.where(valid, dd[sl], 0.0)

        def _compute(si, dl, dd, ro):
            def _edge16(jv, _):
                d2vec = dd[pl.ds(16 * jv, 16)]
                for t in range(16):
                    d2s = d2vec[t]
                    j = 16 * jv + t
                    for k in range(8):
                        sl = pl.ds(16 * k, 16)
                        a = d2s * wr_v[sl] + br_v[sl]
                        w = a / (1.0 + jnp.exp(-a))
                        ro[j, sl] = ro[j, sl] * w
                return 0
            lax.fori_loop(0, CE // 16, _edge16, 0)
            pltpu.sync_copy(ro, acc_sh.at[dl], add=True)

        def _pair(i2, _):
            a = pl.multiple_of(start + (2 * i2) * CE, 8)
            b = pl.multiple_of(start + (2 * i2 + 1) * CE, 8)
            bok = (2 * i2 + 1) < nch
            m1 = pltpu.async_copy(srcp_hbm.at[pl.ds(a, CE)], sidx_v, semm)
            m2 = pltpu.async_copy(dstp_hbm.at[pl.ds(a, CE)], dstl_v, semm)
            m3 = pltpu.async_copy(d2p_hbm.at[pl.ds(a, CE)], d2_v, semm)

            @pl.when(bok)
            def _():
                pltpu.async_copy(srcp_hbm.at[pl.ds(b, CE)], sidx_w, semm)
                pltpu.async_copy(dstp_hbm.at[pl.ds(b, CE)], dstl_w, semm)
                pltpu.async_copy(d2p_hbm.at[pl.ds(b, CE)], d2_w, semm)

            m1.wait()
            m2.wait()
            m3.wait()
            _sanitize(a, sidx_v, dstl_v, d2_v)
            ga = pltpu.async_copy(h_hbm.at[sidx_v], rows_v, sem)

            @pl.when(bok)
            def _():
                pltpu.make_async_copy(srcp_hbm.at[pl.ds(b, CE)], sidx_w,
                                      semm).wait()
                pltpu.make_async_copy(dstp_hbm.at[pl.ds(b, CE)], dstl_w,
                                      semm).wait()
                pltpu.make_async_copy(d2p_hbm.at[pl.ds(b, CE)], d2_w,
                                      semm).wait()
                _sanitize(b, sidx_w, dstl_w, d2_w)
                pltpu.async_copy(h_hbm.at[sidx_w], rows_w, semb)

            ga.wait()
            _compute(sidx_v, dstl_v, d2_v, rows_v)

            @pl.when(bok)
            def _():
                pltpu.make_async_copy(h_hbm.at[sidx_w], rows_w, semb).wait()
                _compute(sidx_w, dstl_w, d2_w, rows_w)
            return 0
        lax.fori_loop(0, (nch + 1) >> 1, _pair, 0)
        plsc.subcore_barrier()

        for k in range(25):
            db = s_idx + 16 * k
            @pl.when(db < ABLK)
            def _():
                pltpu.sync_copy(acc_sh.at[pl.ds(db * 32, 32)],
                                agg_hbm.at[pl.ds(q * QROWS + db * 32, 32)])
        return 0

    lax.fori_loop(0, 2, _round, 0)


def _edge_layer(h, srcp, dstp, d2p, wr, br, cnts):
    mesh = plsc.VectorSubcoreMesh(core_axis_name="c", subcore_axis_name="s")
    f = pl.kernel(
        _edge_layer_body,
        out_type=jax.ShapeDtypeStruct((NP, H), jnp.float32),
        mesh=mesh,
        scratch_types=[
            pltpu.VMEM((16,), jnp.int32),
            pltpu.VMEM((16,), jnp.int32),
            pltpu.VMEM((H,), jnp.float32),
            pltpu.VMEM((H,), jnp.float32),
            pltpu.VMEM((CE,), jnp.int32),
            pltpu.VMEM((CE,), jnp.int32),
            pltpu.VMEM((CE,), jnp.float32),
            pltpu.VMEM((CE, H), jnp.float32),
            pltpu.VMEM((CE,), jnp.int32),
            pltpu.VMEM((CE,), jnp.int32),
            pltpu.VMEM((CE,), jnp.float32),
            pltpu.VMEM((CE, H), jnp.float32),
            pltpu.VMEM_SHARED((AROWS, H), jnp.float32),
            pltpu.SemaphoreType.DMA,
            pltpu.SemaphoreType.DMA,
            pltpu.SemaphoreType.DMA,
        ],
    )
    return f(h, srcp, dstp, d2p, wr, br, cnts)


TBLK = 2048


def _embed_body(x_ref, we_ref, be_ref, out_ref):
    out_ref[...] = x_ref[...] @ we_ref[...] + be_ref[...]


def _embed(x_pad, we, be):
    f_in = x_pad.shape[1]
    return pl.pallas_call(
        _embed_body,
        grid=(pl.cdiv(NP, TBLK),),
        in_specs=[
            pl.BlockSpec((TBLK, f_in), lambda i: (i, 0)),
            pl.BlockSpec((f_in, H), lambda i: (0, 0)),
            pl.BlockSpec((1, H), lambda i: (0, 0)),
        ],
        out_specs=pl.BlockSpec((TBLK, H), lambda i: (i, 0)),
        out_shape=jax.ShapeDtypeStruct((NP, H), jnp.float32),
    )(x_pad, we, be.reshape(1, H))


def _update_body(h_ref, agg_ref, wn_ref, bn_ref, out_ref):
    a = agg_ref[...] @ wn_ref[...] + bn_ref[...]
    out_ref[...] = h_ref[...] + a / (1.0 + jnp.exp(-a))


def _dense_update(h, agg, wn, bn):
    return pl.pallas_call(
        _update_body,
        grid=(pl.cdiv(NP, TBLK),),
        in_specs=[
            pl.BlockSpec((TBLK, H), lambda i: (i, 0)),
            pl.BlockSpec((TBLK, H), lambda i: (i, 0)),
            pl.BlockSpec((H, H), lambda i: (0, 0)),
            pl.BlockSpec((1, H), lambda i: (0, 0)),
        ],
        out_specs=pl.BlockSpec((TBLK, H), lambda i: (i, 0)),
        out_shape=jax.ShapeDtypeStruct((NP, H), jnp.float32),
    )(h, agg, wn, bn.reshape(1, H))


def _pool_mlp_body(h_ref, w1_ref, b1_ref, w2_ref, b2_ref, out_ref, acc_ref):
    i = pl.program_id(0)
    nblk = pl.num_programs(0)

    @pl.when(i == 0)
    def _():
        acc_ref[...] = jnp.zeros_like(acc_ref)

    rows = lax.broadcasted_iota(jnp.int32, (TBLK, H), 0) + i * TBLK
    inq = rows - (rows // QROWS) * QROWS
    m = (inq < QN) & (rows < NP)
    blk = jnp.where(m, h_ref[...], 0.0)
    acc_ref[...] += jnp.sum(blk, axis=0, keepdims=True)

    @pl.when(i == nblk - 1)
    def _():
        pooled = acc_ref[...] * (1.0 / N)
        z = jnp.maximum(pooled @ w1_ref[...] + b1_ref[...], 0.0)
        out_ref[...] = z @ w2_ref[...] + b2_ref[...]


def _pool_mlp(h, w1, b1, w2, b2):
    out_d = w2.shape[1]
    return pl.pallas_call(
        _pool_mlp_body,
        grid=(pl.cdiv(NP, TBLK),),
        in_specs=[
            pl.BlockSpec((TBLK, H), lambda i: (i, 0)),
            pl.BlockSpec(w1.shape, lambda i: (0, 0)),
            pl.BlockSpec((1, out_d), lambda i: (0, 0)),
            pl.BlockSpec(w2.shape, lambda i: (0, 0)),
            pl.BlockSpec((1, out_d), lambda i: (0, 0)),
        ],
        out_specs=pl.BlockSpec((1, out_d), lambda i: (0, 0)),
        out_shape=jax.ShapeDtypeStruct((1, out_d), jnp.float32),
        scratch_shapes=[pltpu.VMEM((1, H), jnp.float32)],
    )(h, w1, b1.reshape(1, -1), w2, b2.reshape(1, -1))


def kernel(x, pos, edge_index, W_emb, b_emb, Wr, br, Wn, bn, W1, b1, W2, b2):
    L = Wr.shape[0]
    zpad = jnp.zeros((C,), jnp.int32)
    srcE = jnp.concatenate([edge_index[0], zpad])
    dstE = jnp.concatenate([edge_index[1], zpad])
    px = pos[:, 0]
    py = pos[:, 1]
    pz = pos[:, 2]
    rowmap = jnp.arange(N) + (jnp.arange(N) // QN) * (QROWS - QN)
    x_pad = jnp.zeros((NP, x.shape[1]), x.dtype).at[rowmap].set(x)

    srcp, dstp, d2p, cnts = _bucket_edges(srcE, dstE, px, py, pz)

    h = _embed(x_pad, W_emb, b_emb)
    for l in range(L):
        agg = _edge_layer(h, srcp, dstp, d2p, Wr[l, 0], br[l], cnts)
        h = _dense_update(h, agg, Wn[l], bn[l])
    return _pool_mlp(h, W1, b1, W2, b2)


# speedup vs baseline: 2.3173x; 2.3173x over previous
import jax
import jax.numpy as jnp
from jax import lax
from jax.experimental import pallas as pl
from jax.experimental.pallas import tpu as pltpu
from jax.experimental.pallas import tpu_sc as plsc

N = 50000
E = 800000
H = 128
NB = 4
QN = N // NB
QROWS = 12544
NP = NB * QROWS
NBLK = QROWS // 128
CAPQ = 262144
CAPH = CAPQ // 2
EP = NB * CAPQ
EPP = EP + 64
C = 128
CE = 112
AROWS = 12512
ABLK = AROWS // 32
EHALF = E // 2
ETILE = E // 32
NBATCH = (ETILE + C - 1) // C


def _iota16():
    return lax.broadcasted_iota(jnp.int32, (16,), 0)


def _prefix16(m):
    iota = _iota16()
    v = jnp.where(m, 1, 0)
    for k in (1, 2, 4, 8):
        sh = v.at[jnp.maximum(iota - k, 0)].get(mode="promise_in_bounds")
        v = v + jnp.where(iota >= k, sh, 0)
    return v


def _bucket_body(srcE, dstE, px, py, pz,
                 srcp_o, dstp_o, d2p_o, cnts_o,
                 sv, dv, d2s, pidx, gx, gy, gz, hx, hy, hz, c16, csm, sem):
    c_idx = lax.axis_index("c")
    s_idx = lax.axis_index("s")
    iota = _iota16()

    @pl.when(s_idx == 0)
    def _():
        for b in range(4):
            csm[b] = 0
    plsc.subcore_barrier()

    tilebase = c_idx * EHALF + s_idx * ETILE

    def _batch(i, _):
        base = pl.multiple_of(tilebase + i * C, 8)
        pltpu.sync_copy(srcE.at[pl.ds(base, C)], sv)
        pltpu.sync_copy(dstE.at[pl.ds(base, C)], dv)
        d1 = pltpu.async_copy(px.at[sv], gx, sem)
        d2_ = pltpu.async_copy(py.at[sv], gy, sem)
        d3 = pltpu.async_copy(pz.at[sv], gz, sem)
        d4 = pltpu.async_copy(px.at[dv], hx, sem)
        d5 = pltpu.async_copy(py.at[dv], hy, sem)
        d6 = pltpu.async_copy(pz.at[dv], hz, sem)
        for d in (d1, d2_, d3, d4, d5, d6):
            d.wait()
        bvs = []
        pcs = []
        tl0 = i * C
        for kv in range(8):
            sl = pl.ds(16 * kv, 16)
            ax = gx[sl] - hx[sl]
            ay = gy[sl] - hy[sl]
            az = gz[sl] - hz[sl]
            d2s[sl] = ax * ax + ay * ay + az * az
            dvv = dv[sl]
            bv = (jnp.where(dvv >= QN, 1, 0)
                  + jnp.where(dvv >= 2 * QN, 1, 0)
                  + jnp.where(dvv >= 3 * QN, 1, 0))
            tl = iota + (tl0 + 16 * kv)
            bv = jnp.where(tl < ETILE, bv, 4)
            bvs.append(bv)
            row = []
            for b in range(4):
                row.append(_prefix16(bv == b)[15])
            pcs.append(row)
        gbase = []
        for b in range(4):
            nb_b = pcs[0][b]
            for kv in range(1, 8):
                nb_b = nb_b + pcs[kv][b]
            old = plsc.fetch_and_add(csm.at[b], nb_b, subcore_id=0)
            gbase.append(b * CAPQ + c_idx * CAPH + old)
        pre = [0, 0, 0, 0]
        for kv in range(8):
            sl = pl.ds(16 * kv, 16)
            bv = bvs[kv]
            pos = jnp.full((16,), EP, jnp.int32)
            for b in range(4):
                m = bv == b
                cs = _prefix16(m)
                pos = jnp.where(m, (gbase[b] + pre[b] - 1) + cs, pos)
                pre[b] = pre[b] + pcs[kv][b]
            pidx[sl] = pos
            svv = sv[sl]
            sv[sl] = svv + 44 * (jnp.where(svv >= QN, 1, 0)
                                 + jnp.where(svv >= 2 * QN, 1, 0)
                                 + jnp.where(svv >= 3 * QN, 1, 0))
        e1 = pltpu.async_copy(sv, srcp_o.at[pidx], sem)
        e2 = pltpu.async_copy(dv, dstp_o.at[pidx], sem)
        e3 = pltpu.async_copy(d2s, d2p_o.at[pidx], sem)
        e1.wait()
        e2.wait()
        e3.wait()
        return 0

    lax.fori_loop(0, NBATCH, _batch, 0)
    plsc.subcore_barrier()

    @pl.when(s_idx == 0)
    def _():
        v = jnp.zeros((16,), jnp.int32)
        for b in range(4):
            v = jnp.where(iota == b, csm[b], v)
        c16[...] = v
        pltpu.sync_copy(c16, cnts_o.at[c_idx])


def _bucket_edges(srcE, dstE, px, py, pz):
    mesh = plsc.VectorSubcoreMesh(core_axis_name="c", subcore_axis_name="s")
    f = pl.kernel(
        _bucket_body,
        out_type=(
            jax.ShapeDtypeStruct((EPP,), jnp.int32),
            jax.ShapeDtypeStruct((EPP,), jnp.int32),
            jax.ShapeDtypeStruct((EPP,), jnp.float32),
            jax.ShapeDtypeStruct((2, 16), jnp.int32),
        ),
        mesh=mesh,
        scratch_types=[
            pltpu.VMEM((C,), jnp.int32),
            pltpu.VMEM((C,), jnp.int32),
            pltpu.VMEM((C,), jnp.float32),
            pltpu.VMEM((C,), jnp.int32),
            pltpu.VMEM((C,), jnp.float32),
            pltpu.VMEM((C,), jnp.float32),
            pltpu.VMEM((C,), jnp.float32),
            pltpu.VMEM((C,), jnp.float32),
            pltpu.VMEM((C,), jnp.float32),
            pltpu.VMEM((C,), jnp.float32),
            pltpu.VMEM((16,), jnp.int32),
            pltpu.SMEM((8,), jnp.int32),
            pltpu.SemaphoreType.DMA,
        ],
    )
    return f(srcE, dstE, px, py, pz)


def _edge_layer_body(h_hbm, srcp_hbm, dstp_hbm, d2p_hbm, wr_hbm, br_hbm,
                     cnts_hbm, agg_hbm,
                     c0_v, c1_v, wr_v, br_v, sidx_v, dstl_v, d2_v, rows_v,
                     acc_sh, sem):
    c_idx = lax.axis_index("c")
    s_idx = lax.axis_index("s")
    pltpu.sync_copy(cnts_hbm.at[0], c0_v)
    pltpu.sync_copy(cnts_hbm.at[1], c1_v)
    pltpu.sync_copy(wr_hbm, wr_v)
    pltpu.sync_copy(br_hbm, br_v)
    c0 = c0_v[...]
    c1 = c1_v[...]
    iota = _iota16()
    wrs = [wr_v[pl.ds(16 * k, 16)] for k in range(8)]
    brs = [br_v[pl.ds(16 * k, 16)] for k in range(8)]
    jsel = s_idx >> 3
    t8 = s_idx & 7

    for r in range(2):
        q = c_idx + 2 * r
        cnt0 = jnp.where(c_idx == 0, c0[2 * r], c0[2 * r + 1])
        cnt1 = jnp.where(c_idx == 0, c1[2 * r], c1[2 * r + 1])
        cnt = jnp.where(jsel == 0, cnt0, cnt1)
        rb = q * CAPQ + jsel * CAPH
        qnode = q * QN
        share = ((cnt + 1023) >> 10) << 7
        start = rb + t8 * share
        tend = jnp.minimum(start + share, rb + cnt)
        nch = jnp.maximum(tend - start, 0) >> 7

        plsc.subcore_barrier()

        def _zrow(j, _):
            for k in range(8):
                rows_v[j, pl.ds(16 * k, 16)] = jnp.zeros((16,), jnp.float32)
            return 0
        lax.fori_loop(0, 128, _zrow, 0)
        for k in range(7):
            b = s_idx + 16 * k
            @pl.when(b < NBLK)
            def _():
                pltpu.sync_copy(rows_v, acc_sh.at[pl.ds(b * 128, 128)])
        plsc.subcore_barrier()

        def _chunk(i, _):
            base = pl.multiple_of(start + i * C, C)
            pltpu.sync_copy(srcp_hbm.at[pl.ds(base, C)], sidx_v)
            pltpu.sync_copy(dstp_hbm.at[pl.ds(base, C)], dstl_v)
            pltpu.sync_copy(d2p_hbm.at[pl.ds(base, C)], d2_v)
            for kv in range(8):
                sl = pl.ds(16 * kv, 16)
                posv = iota + (base + 16 * kv)
                valid = posv < tend
                sidx_v[sl] = jnp.clip(sidx_v[sl], 0, NP - 1)
                dv = jnp.clip(dstl_v[sl] - qnode, 0, QN - 1)
                dstl_v[sl] = jnp.where(valid, dv, QN)
                d2_v[sl] = jnp.where(valid, d2_v[sl], 0.0)
            pltpu.async_copy(h_hbm.at[sidx_v], rows_v, sem).wait()

            def _edge16(jv, _):
                d2vec = d2_v[pl.ds(16 * jv, 16)]
                for t in range(16):
                    d2s = d2vec[t]
                    j = 16 * jv + t
                    for k in range(8):
                        sl = pl.ds(16 * k, 16)
                        a = d2s * wrs[k] + brs[k]
                        w = a / (1.0 + jnp.exp(-a))
                        rows_v[j, sl] = rows_v[j, sl] * w
                return 0
            lax.fori_loop(0, C // 16, _edge16, 0)
            pltpu.sync_copy(rows_v, acc_sh.at[dstl_v], add=True)
            return 0
        lax.fori_loop(0, nch, _chunk, 0)
        plsc.subcore_barrier()

        for k in range(7):
            b = s_idx + 16 * k
            @pl.when(b < NBLK)
            def _():
                pltpu.sync_copy(acc_sh.at[pl.ds(b * 128, 128)],
                                agg_hbm.at[pl.ds(q * QROWS + b * 128, 128)])


def _edge_layer(h, srcp, dstp, d2p, wr, br, cnts):
    mesh = plsc.VectorSubcoreMesh(core_axis_name="c", subcore_axis_name="s")
    f = pl.kernel(
        _edge_layer_body,
        out_type=jax.ShapeDtypeStruct((NP, H), jnp.float32),
        mesh=mesh,
        scratch_types=[
            pltpu.VMEM((16,), jnp.int32),
            pltpu.VMEM((16,), jnp.int32),
            pltpu.VMEM((H,), jnp.float32),
            pltpu.VMEM((H,), jnp.float32),
            pltpu.VMEM((C,), jnp.int32),
            pltpu.VMEM((C,), jnp.int32),
            pltpu.VMEM((C,), jnp.float32),
            pltpu.VMEM((C, H), jnp.float32),
            pltpu.VMEM_SHARED((QROWS, H), jnp.float32),
            pltpu.SemaphoreType.DMA,
        ],
    )
    return f(h, srcp, dstp, d2p, wr, br, cnts)


TBLK = 2048


def _embed_body(x_ref, we_ref, be_ref, out_ref):
    out_ref[...] = x_ref[...] @ we_ref[...] + be_ref[...]


def _embed(x_pad, we, be):
    f_in = x_pad.shape[1]
    return pl.pallas_call(
        _embed_body,
        grid=(pl.cdiv(NP, TBLK),),
        in_specs=[
            pl.BlockSpec((TBLK, f_in), lambda i: (i, 0)),
            pl.BlockSpec((f_in, H), lambda i: (0, 0)),
            pl.BlockSpec((1, H), lambda i: (0, 0)),
        ],
        out_specs=pl.BlockSpec((TBLK, H), lambda i: (i, 0)),
        out_shape=jax.ShapeDtypeStruct((NP, H), jnp.float32),
    )(x_pad, we, be.reshape(1, H))


def _update_body(h_ref, agg_ref, wn_ref, bn_ref, out_ref):
    a = agg_ref[...] @ wn_ref[...] + bn_ref[...]
    out_ref[...] = h_ref[...] + a / (1.0 + jnp.exp(-a))


def _dense_update(h, agg, wn, bn):
    return pl.pallas_call(
        _update_body,
        grid=(pl.cdiv(NP, TBLK),),
        in_specs=[
            pl.BlockSpec((TBLK, H), lambda i: (i, 0)),
            pl.BlockSpec((TBLK, H), lambda i: (i, 0)),
            pl.BlockSpec((H, H), lambda i: (0, 0)),
            pl.BlockSpec((1, H), lambda i: (0, 0)),
        ],
        out_specs=pl.BlockSpec((TBLK, H), lambda i: (i, 0)),
        out_shape=jax.ShapeDtypeStruct((NP, H), jnp.float32),
    )(h, agg, wn, bn.reshape(1, H))


def _pool_mlp_body(h_ref, w1_ref, b1_ref, w2_ref, b2_ref, out_ref, acc_ref):
    i = pl.program_id(0)
    nblk = pl.num_programs(0)

    @pl.when(i == 0)
    def _():
        acc_ref[...] = jnp.zeros_like(acc_ref)

    rows = lax.broadcasted_iota(jnp.int32, (TBLK, H), 0) + i * TBLK
    inq = rows - (rows // QROWS) * QROWS
    m = (inq < QN) & (rows < NP)
    blk = jnp.where(m, h_ref[...], 0.0)
    acc_ref[...] += jnp.sum(blk, axis=0, keepdims=True)

    @pl.when(i == nblk - 1)
    def _():
        pooled = acc_ref[...] * (1.0 / N)
        z = jnp.maximum(pooled @ w1_ref[...] + b1_ref[...], 0.0)
        out_ref[...] = z @ w2_ref[...] + b2_ref[...]


def _pool_mlp(h, w1, b1, w2, b2):
    out_d = w2.shape[1]
    return pl.pallas_call(
        _pool_mlp_body,
        grid=(pl.cdiv(NP, TBLK),),
        in_specs=[
            pl.BlockSpec((TBLK, H), lambda i: (i, 0)),
            pl.BlockSpec(w1.shape, lambda i: (0, 0)),
            pl.BlockSpec((1, out_d), lambda i: (0, 0)),
            pl.BlockSpec(w2.shape, lambda i: (0, 0)),
            pl.BlockSpec((1, out_d), lambda i: (0, 0)),
        ],
        out_specs=pl.BlockSpec((1, out_d), lambda i: (0, 0)),
        out_shape=jax.ShapeDtypeStruct((1, out_d), jnp.float32),
        scratch_shapes=[pltpu.VMEM((1, H), jnp.float32)],
    )(h, w1, b1.reshape(1, -1), w2, b2.reshape(1, -1))


def kernel(x, pos, edge_index, W_emb, b_emb, Wr, br, Wn, bn, W1, b1, W2, b2):
    L = Wr.shape[0]
    zpad = jnp.zeros((C,), jnp.int32)
    srcE = jnp.concatenate([edge_index[0], zpad])
    dstE = jnp.concatenate([edge_index[1], zpad])
    px = pos[:, 0]
    py = pos[:, 1]
    pz = pos[:, 2]
    rowmap = jnp.arange(N) + (jnp.arange(N) // QN) * (QROWS - QN)
    x_pad = jnp.zeros((NP, x.shape[1]), x.dtype).at[rowmap].set(x)

    srcp, dstp, d2p, cnts = _bucket_edges(srcE, dstE, px, py, pz)

    h = _embed(x_pad, W_emb, b_emb)
    for l in range(L):
        agg = _edge_layer(h, srcp, dstp, d2p, Wr[l, 0], br[l], cnts)
        h = _dense_update(h, agg, Wn[l], bn[l])
    return _pool_mlp(h, W1, b1, W2, b2)


# speedup vs baseline: 2.4179x; 1.0434x over previous
import jax
import jax.numpy as jnp
from jax import lax
from jax.experimental import pallas as pl
from jax.experimental.pallas import tpu as pltpu
from jax.experimental.pallas import tpu_sc as plsc

N = 50000
E = 800000
H = 128
NB = 4
QN = N // NB
QROWS = 12544
NP = NB * QROWS
NBLK = QROWS // 128
CAPQ = 262144
CAPH = CAPQ // 2
EP = NB * CAPQ
EPP = EP + 64
C = 128
CE = 112
AROWS = 12512
ABLK = AROWS // 32
EHALF = E // 2
ETILE = E // 32
NBATCH = (ETILE + C - 1) // C


def _iota16():
    return lax.broadcasted_iota(jnp.int32, (16,), 0)


def _prefix16(m):
    iota = _iota16()
    v = jnp.where(m, 1, 0)
    for k in (1, 2, 4, 8):
        sh = v.at[jnp.maximum(iota - k, 0)].get(mode="promise_in_bounds")
        v = v + jnp.where(iota >= k, sh, 0)
    return v


def _bucket_body(srcE, dstE, px, py, pz,
                 srcp_o, dstp_o, d2p_o, cnts_o,
                 sv, dv, d2s, pidx, gx, gy, gz, hx, hy, hz, c16, csm, sem):
    c_idx = lax.axis_index("c")
    s_idx = lax.axis_index("s")
    iota = _iota16()

    @pl.when(s_idx == 0)
    def _():
        for b in range(4):
            csm[b] = 0
    plsc.subcore_barrier()

    tilebase = c_idx * EHALF + s_idx * ETILE

    def _batch(i, _):
        base = pl.multiple_of(tilebase + i * C, 8)
        pltpu.sync_copy(srcE.at[pl.ds(base, C)], sv)
        pltpu.sync_copy(dstE.at[pl.ds(base, C)], dv)
        d1 = pltpu.async_copy(px.at[sv], gx, sem)
        d2_ = pltpu.async_copy(py.at[sv], gy, sem)
        d3 = pltpu.async_copy(pz.at[sv], gz, sem)
        d4 = pltpu.async_copy(px.at[dv], hx, sem)
        d5 = pltpu.async_copy(py.at[dv], hy, sem)
        d6 = pltpu.async_copy(pz.at[dv], hz, sem)
        for d in (d1, d2_, d3, d4, d5, d6):
            d.wait()
        bvs = []
        pcs = []
        tl0 = i * C
        for kv in range(8):
            sl = pl.ds(16 * kv, 16)
            ax = gx[sl] - hx[sl]
            ay = gy[sl] - hy[sl]
            az = gz[sl] - hz[sl]
            d2s[sl] = ax * ax + ay * ay + az * az
            dvv = dv[sl]
            bv = (jnp.where(dvv >= QN, 1, 0)
                  + jnp.where(dvv >= 2 * QN, 1, 0)
                  + jnp.where(dvv >= 3 * QN, 1, 0))
            tl = iota + (tl0 + 16 * kv)
            bv = jnp.where(tl < ETILE, bv, 4)
            bvs.append(bv)
            row = []
            for b in range(4):
                row.append(_prefix16(bv == b)[15])
            pcs.append(row)
        gbase = []
        for b in range(4):
            nb_b = pcs[0][b]
            for kv in range(1, 8):
                nb_b = nb_b + pcs[kv][b]
            old = plsc.fetch_and_add(csm.at[b], nb_b, subcore_id=0)
            gbase.append(b * CAPQ + c_idx * CAPH + old)
        pre = [0, 0, 0, 0]
        for kv in range(8):
            sl = pl.ds(16 * kv, 16)
            bv = bvs[kv]
            pos = jnp.full((16,), EP, jnp.int32)
            for b in range(4):
                m = bv == b
                cs = _prefix16(m)
                pos = jnp.where(m, (gbase[b] + pre[b] - 1) + cs, pos)
                pre[b] = pre[b] + pcs[kv][b]
            pidx[sl] = pos
            svv = sv[sl]
            sv[sl] = svv + 44 * (jnp.where(svv >= QN, 1, 0)
                                 + jnp.where(svv >= 2 * QN, 1, 0)
                                 + jnp.where(svv >= 3 * QN, 1, 0))
        e1 = pltpu.async_copy(sv, srcp_o.at[pidx], sem)
        e2 = pltpu.async_copy(dv, dstp_o.at[pidx], sem)
        e3 = pltpu.async_copy(d2s, d2p_o.at[pidx], sem)
        e1.wait()
        e2.wait()
        e3.wait()
        return 0

    lax.fori_loop(0, NBATCH, _batch, 0)
    plsc.subcore_barrier()

    @pl.when(s_idx == 0)
    def _():
        v = jnp.zeros((16,), jnp.int32)
        for b in range(4):
            v = jnp.where(iota == b, csm[b], v)
        c16[...] = v
        pltpu.sync_copy(c16, cnts_o.at[c_idx])


def _bucket_edges(srcE, dstE, px, py, pz):
    mesh = plsc.VectorSubcoreMesh(core_axis_name="c", subcore_axis_name="s")
    f = pl.kernel(
        _bucket_body,
        out_type=(
            jax.ShapeDtypeStruct((EPP,), jnp.int32),
            jax.ShapeDtypeStruct((EPP,), jnp.int32),
            jax.ShapeDtypeStruct((EPP,), jnp.float32),
            jax.ShapeDtypeStruct((2, 16), jnp.int32),
        ),
        mesh=mesh,
        scratch_types=[
            pltpu.VMEM((C,), jnp.int32),
            pltpu.VMEM((C,), jnp.int32),
            pltpu.VMEM((C,), jnp.float32),
            pltpu.VMEM((C,), jnp.int32),
            pltpu.VMEM((C,), jnp.float32),
            pltpu.VMEM((C,), jnp.float32),
            pltpu.VMEM((C,), jnp.float32),
            pltpu.VMEM((C,), jnp.float32),
            pltpu.VMEM((C,), jnp.float32),
            pltpu.VMEM((C,), jnp.float32),
            pltpu.VMEM((16,), jnp.int32),
            pltpu.SMEM((8,), jnp.int32),
            pltpu.SemaphoreType.DMA,
        ],
    )
    return f(srcE, dstE, px, py, pz)


def _edge_layer_body(h_hbm, srcp_hbm, dstp_hbm, d2p_hbm, wr_hbm, br_hbm,
                     cnts_hbm, agg_hbm,
                     c0_v, c1_v, wr_v, br_v, sidx_v, dstl_v, d2_v, rows_v,
                     acc_sh, sem, semm):
    c_idx = lax.axis_index("c")
    s_idx = lax.axis_index("s")
    pltpu.sync_copy(cnts_hbm.at[0], c0_v)
    pltpu.sync_copy(cnts_hbm.at[1], c1_v)
    pltpu.sync_copy(wr_hbm, wr_v)
    pltpu.sync_copy(br_hbm, br_v)
    c0 = c0_v[...]
    c1 = c1_v[...]
    iota = _iota16()
    wrs = [wr_v[pl.ds(16 * k, 16)] for k in range(8)]
    brs = [br_v[pl.ds(16 * k, 16)] for k in range(8)]
    jsel = s_idx >> 3
    t8 = s_idx & 7

    for r in range(2):
        q = c_idx + 2 * r
        cnt0 = jnp.where(c_idx == 0, c0[2 * r], c0[2 * r + 1])
        cnt1 = jnp.where(c_idx == 0, c1[2 * r], c1[2 * r + 1])
        cnt = jnp.where(jsel == 0, cnt0, cnt1)
        rb = q * CAPQ + jsel * CAPH
        qnode = q * QN
        share = ((cnt + 1023) >> 10) << 7
        start = rb + t8 * share
        tend = jnp.minimum(start + share, rb + cnt)
        nch = jnp.maximum(tend - start, 0) >> 7

        plsc.subcore_barrier()

        def _zrow(j, _):
            for k in range(8):
                rows_v[j, pl.ds(16 * k, 16)] = jnp.zeros((16,), jnp.float32)
            return 0
        lax.fori_loop(0, 128, _zrow, 0)
        for k in range(7):
            b = s_idx + 16 * k
            @pl.when(b < NBLK)
            def _():
                pltpu.sync_copy(rows_v, acc_sh.at[pl.ds(b * 128, 128)])
        plsc.subcore_barrier()

        def _chunk(i, _):
            base = pl.multiple_of(start + i * C, C)
            m1 = pltpu.async_copy(srcp_hbm.at[pl.ds(base, C)], sidx_v, semm)
            m2 = pltpu.async_copy(dstp_hbm.at[pl.ds(base, C)], dstl_v, semm)
            m3 = pltpu.async_copy(d2p_hbm.at[pl.ds(base, C)], d2_v, semm)
            m1.wait()
            m2.wait()
            m3.wait()
            for kv in range(8):
                sl = pl.ds(16 * kv, 16)
                posv = iota + (base + 16 * kv)
                valid = posv < tend
                sidx_v[sl] = jnp.clip(sidx_v[sl], 0, NP - 1)
                dv = jnp.clip(dstl_v[sl] - qnode, 0, QN - 1)
                dstl_v[sl] = jnp.where(valid, dv, QN)
                d2_v[sl] = jnp.where(valid, d2_v[sl], 0.0)
            pltpu.async_copy(h_hbm.at[sidx_v], rows_v, sem).wait()

            def _edge16(jv, _):
                d2vec = d2_v[pl.ds(16 * jv, 16)]
                for t in range(16):
                    d2s = d2vec[t]
                    j = 16 * jv + t
                    for k in range(8):
                        sl = pl.ds(16 * k, 16)
                        a = d2s * wrs[k] + brs[k]
                        w = a / (1.0 + jnp.exp(-a))
                        rows_v[j, sl] = rows_v[j, sl] * w
                return 0
            lax.fori_loop(0, C // 16, _edge16, 0)
            pltpu.sync_copy(rows_v, acc_sh.at[dstl_v], add=True)
            return 0
        lax.fori_loop(0, nch, _chunk, 0)
        plsc.subcore_barrier()

        for k in range(7):
            b = s_idx + 16 * k
            @pl.when(b < NBLK)
            def _():
                pltpu.sync_copy(acc_sh.at[pl.ds(b * 128, 128)],
                                agg_hbm.at[pl.ds(q * QROWS + b * 128, 128)])


def _edge_layer(h, srcp, dstp, d2p, wr, br, cnts):
    mesh = plsc.VectorSubcoreMesh(core_axis_name="c", subcore_axis_name="s")
    f = pl.kernel(
        _edge_layer_body,
        out_type=jax.ShapeDtypeStruct((NP, H), jnp.float32),
        mesh=mesh,
        scratch_types=[
            pltpu.VMEM((16,), jnp.int32),
            pltpu.VMEM((16,), jnp.int32),
            pltpu.VMEM((H,), jnp.float32),
            pltpu.VMEM((H,), jnp.float32),
            pltpu.VMEM((C,), jnp.int32),
            pltpu.VMEM((C,), jnp.int32),
            pltpu.VMEM((C,), jnp.float32),
            pltpu.VMEM((C, H), jnp.float32),
            pltpu.VMEM_SHARED((QROWS, H), jnp.float32),
            pltpu.SemaphoreType.DMA,
            pltpu.SemaphoreType.DMA,
        ],
    )
    return f(h, srcp, dstp, d2p, wr, br, cnts)


TBLK = 2048


def _embed_body(x_ref, we_ref, be_ref, out_ref):
    out_ref[...] = x_ref[...] @ we_ref[...] + be_ref[...]


def _embed(x_pad, we, be):
    f_in = x_pad.shape[1]
    return pl.pallas_call(
        _embed_body,
        grid=(pl.cdiv(NP, TBLK),),
        in_specs=[
            pl.BlockSpec((TBLK, f_in), lambda i: (i, 0)),
            pl.BlockSpec((f_in, H), lambda i: (0, 0)),
            pl.BlockSpec((1, H), lambda i: (0, 0)),
        ],
        out_specs=pl.BlockSpec((TBLK, H), lambda i: (i, 0)),
        out_shape=jax.ShapeDtypeStruct((NP, H), jnp.float32),
    )(x_pad, we, be.reshape(1, H))


def _update_body(h_ref, agg_ref, wn_ref, bn_ref, out_ref):
    a = agg_ref[...] @ wn_ref[...] + bn_ref[...]
    out_ref[...] = h_ref[...] + a / (1.0 + jnp.exp(-a))


def _dense_update(h, agg, wn, bn):
    return pl.pallas_call(
        _update_body,
        grid=(pl.cdiv(NP, TBLK),),
        in_specs=[
            pl.BlockSpec((TBLK, H), lambda i: (i, 0)),
            pl.BlockSpec((TBLK, H), lambda i: (i, 0)),
            pl.BlockSpec((H, H), lambda i: (0, 0)),
            pl.BlockSpec((1, H), lambda i: (0, 0)),
        ],
        out_specs=pl.BlockSpec((TBLK, H), lambda i: (i, 0)),
        out_shape=jax.ShapeDtypeStruct((NP, H), jnp.float32),
    )(h, agg, wn, bn.reshape(1, H))


def _pool_mlp_body(h_ref, w1_ref, b1_ref, w2_ref, b2_ref, out_ref, acc_ref):
    i = pl.program_id(0)
    nblk = pl.num_programs(0)

    @pl.when(i == 0)
    def _():
        acc_ref[...] = jnp.zeros_like(acc_ref)

    rows = lax.broadcasted_iota(jnp.int32, (TBLK, H), 0) + i * TBLK
    inq = rows - (rows // QROWS) * QROWS
    m = (inq < QN) & (rows < NP)
    blk = jnp.where(m, h_ref[...], 0.0)
    acc_ref[...] += jnp.sum(blk, axis=0, keepdims=True)

    @pl.when(i == nblk - 1)
    def _():
        pooled = acc_ref[...] * (1.0 / N)
        z = jnp.maximum(pooled @ w1_ref[...] + b1_ref[...], 0.0)
        out_ref[...] = z @ w2_ref[...] + b2_ref[...]


def _pool_mlp(h, w1, b1, w2, b2):
    out_d = w2.shape[1]
    return pl.pallas_call(
        _pool_mlp_body,
        grid=(pl.cdiv(NP, TBLK),),
        in_specs=[
            pl.BlockSpec((TBLK, H), lambda i: (i, 0)),
            pl.BlockSpec(w1.shape, lambda i: (0, 0)),
            pl.BlockSpec((1, out_d), lambda i: (0, 0)),
            pl.BlockSpec(w2.shape, lambda i: (0, 0)),
            pl.BlockSpec((1, out_d), lambda i: (0, 0)),
        ],
        out_specs=pl.BlockSpec((1, out_d), lambda i: (0, 0)),
        out_shape=jax.ShapeDtypeStruct((1, out_d), jnp.float32),
        scratch_shapes=[pltpu.VMEM((1, H), jnp.float32)],
    )(h, w1, b1.reshape(1, -1), w2, b2.reshape(1, -1))


def kernel(x, pos, edge_index, W_emb, b_emb, Wr, br, Wn, bn, W1, b1, W2, b2):
    L = Wr.shape[0]
    zpad = jnp.zeros((C,), jnp.int32)
    srcE = jnp.concatenate([edge_index[0], zpad])
    dstE = jnp.concatenate([edge_index[1], zpad])
    px = pos[:, 0]
    py = pos[:, 1]
    pz = pos[:, 2]
    rowmap = jnp.arange(N) + (jnp.arange(N) // QN) * (QROWS - QN)
    x_pad = jnp.zeros((NP, x.shape[1]), x.dtype).at[rowmap].set(x)

    srcp, dstp, d2p, cnts = _bucket_edges(srcE, dstE, px, py, pz)

    h = _embed(x_pad, W_emb, b_emb)
    for l in range(L):
        agg = _edge_layer(h, srcp, dstp, d2p, Wr[l, 0], br[l], cnts)
        h = _dense_update(h, agg, Wn[l], bn[l])
    return _pool_mlp(h, W1, b1, W2, b2)
